# Initial kernel scaffold; baseline (speedup 1.0000x reference)
import jax
import jax.numpy as jnp
from jax.experimental import pallas as pl


def kernel(pos, vel, edge_index):
    raise NotImplementedError("write your pallas kernel here")


# speedup vs baseline: 56.9327x; 56.9327x over previous
import functools

import jax
import jax.numpy as jnp
from jax import lax
from jax.experimental import pallas as pl
from jax.experimental.pallas import tpu as pltpu
from jax.experimental.pallas import tpu_sc as plsc

MIN_DIST = 5.0
L = 16
NC = 2
NS = 16
NW = NC * NS
CHUNK = 2048
KSUB = CHUNK // 128


def _rsqrt(x):
    r = jnp.ones_like(x)
    xs = x
    for p in (32, 16, 8, 4, 2, 1):
        up = jnp.float32(2.0 ** (2 * p))
        cond = xs < jnp.float32(2.0 ** (1 - 2 * p))
        xs = jnp.where(cond, xs * up, xs)
        r = jnp.where(cond, r * jnp.float32(2.0 ** p), r)
        dn = jnp.float32(2.0 ** (-2 * p))
        cond2 = xs >= jnp.float32(2.0 ** (2 * p - 1))
        xs = jnp.where(cond2, xs * dn, xs)
        r = jnp.where(cond2, r * jnp.float32(2.0 ** (-p)), r)
    y = 2.007 - xs * (1.364 - 0.357 * xs)
    for _ in range(3):
        y = y * (1.5 - 0.5 * xs * y * y)
    return r * y


def _edge_kernel(n, e):
    chunks_total = e // CHUNK
    base_chunks = chunks_total // NW
    extra = chunks_total - base_chunks * NW
    tile_rows = (n // NS) & ~7
    tail_rows = n - NS * tile_rows

    mesh = plsc.VectorSubcoreMesh(core_axis_name="c", subcore_axis_name="s")

    @functools.partial(
        pl.kernel,
        out_type=jax.ShapeDtypeStruct((NC * 5 * n,), jnp.float32),
        mesh=mesh,
        scratch_types=dict(
            srcv=pltpu.VMEM((CHUNK,), jnp.int32),
            dstv=pltpu.VMEM((KSUB, 128), jnp.int32),
            jxv=pltpu.VMEM((CHUNK,), jnp.float32),
            jyv=pltpu.VMEM((CHUNK,), jnp.float32),
            jvxv=pltpu.VMEM((CHUNK,), jnp.float32),
            jvyv=pltpu.VMEM((CHUNK,), jnp.float32),
            ixv=pltpu.VMEM((CHUNK,), jnp.float32),
            iyv=pltpu.VMEM((CHUNK,), jnp.float32),
            cxv=pltpu.VMEM((CHUNK,), jnp.float32),
            cyv=pltpu.VMEM((CHUNK,), jnp.float32),
            sxv=pltpu.VMEM((CHUNK,), jnp.float32),
            syv=pltpu.VMEM((CHUNK,), jnp.float32),
            onesv=pltpu.VMEM((CHUNK,), jnp.float32),
            tabs=[pltpu.VMEM_SHARED((n,), jnp.float32) for _ in range(4)],
            accs=[pltpu.VMEM_SHARED((n,), jnp.float32) for _ in range(5)],
            sem_g=pltpu.SemaphoreType.DMA,
            sem_s=pltpu.SemaphoreType.DMA,
        ),
    )
    def edge_kernel(hx_hbm, hy_hbm, hvx_hbm, hvy_hbm, src_hbm, dst_hbm,
                    part_hbm, srcv, dstv, jxv, jyv, jvxv, jvyv, ixv, iyv,
                    cxv, cyv, sxv, syv, onesv, tabs, accs, sem_g, sem_s):
        cid = lax.axis_index("c")
        sid = lax.axis_index("s")
        wid = sid * NC + cid

        def init_body(g, carry):
            sl = pl.ds(g * L, L)
            onesv[sl] = jnp.full((L,), 1.0, jnp.float32)
            cxv[sl] = jnp.zeros((L,), jnp.float32)
            return carry
        lax.fori_loop(0, CHUNK // L, init_body, 0)

        def _blocks(r0, nrows):
            off = 0
            while off < nrows:
                sz = min(CHUNK, nrows - off)
                yield r0 + off, sz
                off += sz

        def stage(r0, nrows):
            for t, src in zip(tabs, (hx_hbm, hy_hbm, hvx_hbm, hvy_hbm)):
                for o, sz in _blocks(r0, nrows):
                    pltpu.sync_copy(src.at[pl.ds(o, sz)], jxv.at[pl.ds(0, sz)])
                    pltpu.sync_copy(jxv.at[pl.ds(0, sz)], t.at[pl.ds(o, sz)])

        def zero_accs(r0, nrows):
            nfull = nrows // CHUNK
            rem = nrows - nfull * CHUNK
            for a in accs:
                for b in range(nfull):
                    pltpu.sync_copy(cxv.at[pl.ds(0, CHUNK)],
                                    a.at[pl.ds(r0 + b * CHUNK, CHUNK)])
                if rem:
                    pltpu.sync_copy(cxv.at[pl.ds(0, rem)],
                                    a.at[pl.ds(r0 + nfull * CHUNK, rem)])

        r0 = sid * tile_rows
        stage(r0, tile_rows)
        zero_accs(r0, tile_rows)

        @pl.when(sid == NS - 1)
        def _():
            if tail_rows:
                stage(NS * tile_rows, tail_rows)
                zero_accs(NS * tile_rows, tail_rows)

        plsc.subcore_barrier()

        n_chunks = jnp.where(wid < extra, base_chunks + 1, base_chunks)

        def chunk_body(t, carry):
            ck = wid + t * NW
            pltpu.sync_copy(src_hbm.at[pl.ds(ck * CHUNK, CHUNK)], srcv)
            pltpu.sync_copy(dst_hbm.at[pl.ds(ck * KSUB, KSUB)], dstv)

            def gather_body(j, c2):
                sl = pl.ds(j * 128, 128)
                sidx = srcv.at[sl]
                didx = dstv.at[j]
                hs = [
                    pltpu.async_copy(tabs[0].at[sidx], jxv.at[sl], sem_g),
                    pltpu.async_copy(tabs[1].at[sidx], jyv.at[sl], sem_g),
                    pltpu.async_copy(tabs[2].at[sidx], jvxv.at[sl], sem_g),
                    pltpu.async_copy(tabs[3].at[sidx], jvyv.at[sl], sem_g),
                    pltpu.async_copy(tabs[0].at[didx], ixv.at[sl], sem_g),
                    pltpu.async_copy(tabs[1].at[didx], iyv.at[sl], sem_g),
                ]
                for hd in hs:
                    hd.wait()
                return c2
            lax.fori_loop(0, KSUB, gather_body, 0)

            def group_body(g, c2):
                sl = pl.ds(g * L, L)
                jx = jxv[sl]
                jy = jyv[sl]
                dx = jx - ixv[sl]
                dy = jy - iyv[sl]
                sq = dx * dx + dy * dy
                sqs = jnp.where(sq > 0.0, sq, 1.0)
                inv = _rsqrt(sqs)
                norm = sqs * inv
                eg = jnp.exp(10.0 * norm - 10.0 * MIN_DIST)
                gate = 1.0 / (1.0 + eg)
                w = gate * inv
                cxv[sl] = -(w * dx)
                cyv[sl] = -(w * dy)
                sxv[sl] = jx * (1.0 / 30.0) + jvxv[sl]
                syv[sl] = jy * (1.0 / 30.0) + jvyv[sl]
                return c2
            lax.fori_loop(0, CHUNK // L, group_body, 0)

            def scatter_body(j, c2):
                sl = pl.ds(j * 128, 128)
                idx = dstv.at[j]
                hs = [
                    pltpu.async_copy(cxv.at[sl], accs[0].at[idx], sem_s,
                                     add=True),
                    pltpu.async_copy(cyv.at[sl], accs[1].at[idx], sem_s,
                                     add=True),
                    pltpu.async_copy(sxv.at[sl], accs[2].at[idx], sem_s,
                                     add=True),
                    pltpu.async_copy(syv.at[sl], accs[3].at[idx], sem_s,
                                     add=True),
                    pltpu.async_copy(onesv.at[sl], accs[4].at[idx], sem_s,
                                     add=True),
                ]
                for hd in hs:
                    hd.wait()
                return c2
            lax.fori_loop(0, KSUB, scatter_body, 0)
            return carry
        lax.fori_loop(0, n_chunks, chunk_body, 0)

        plsc.subcore_barrier()

        def copy_out(r0, nrows):
            for c, a in enumerate(accs):
                off = cid * 5 * n + c * n
                for o, sz in _blocks(r0, nrows):
                    pltpu.sync_copy(a.at[pl.ds(o, sz)], jxv.at[pl.ds(0, sz)])
                    pltpu.sync_copy(jxv.at[pl.ds(0, sz)],
                                    part_hbm.at[pl.ds(off + o, sz)])

        copy_out(r0, tile_rows)

        @pl.when(sid == NS - 1)
        def _():
            if tail_rows:
                copy_out(NS * tile_rows, tail_rows)

    return edge_kernel


def _combine_kernel(n):
    base = (n // NW) & ~7
    tail = n - NW * base

    mesh = plsc.VectorSubcoreMesh(core_axis_name="c", subcore_axis_name="s")

    def _combine_rows(part_hbm, px_hbm, py_hbm, vx_hbm, vy_hbm,
                      ox_hbm, oy_hbm, bufs, r0, nrows):
        (a0, a1, s0, s1, c0, b0, b1, t0, t1, c1,
         pxv, pyv, vxv, vyv, oxv, oyv) = bufs
        parts = (a0, a1, s0, s1, c0, b0, b1, t0, t1, c1)
        for i, dst in enumerate(parts):
            pltpu.sync_copy(part_hbm.at[pl.ds(i * n + r0, nrows)], dst)
        pltpu.sync_copy(px_hbm.at[pl.ds(r0, nrows)], pxv)
        pltpu.sync_copy(py_hbm.at[pl.ds(r0, nrows)], pyv)
        pltpu.sync_copy(vx_hbm.at[pl.ds(r0, nrows)], vxv)
        pltpu.sync_copy(vy_hbm.at[pl.ds(r0, nrows)], vyv)

        def group_body(g, carry):
            sl = pl.ds(g * L, L)
            ax = a0[sl] + b0[sl]
            ay = a1[sl] + b1[sl]
            sx = s0[sl] + t0[sl]
            sy = s1[sl] + t1[sl]
            cnt = c0[sl] + c1[sl]
            den = jnp.maximum(cnt, 1.0)
            oxv[sl] = 5.0 * ax + (sx - cnt * (pxv[sl] * (1.0 / 30.0)
                                              + vxv[sl])) / den
            oyv[sl] = 5.0 * ay + (sy - cnt * (pyv[sl] * (1.0 / 30.0)
                                              + vyv[sl])) / den
            return carry
        lax.fori_loop(0, nrows // L, group_body, 0)

        pltpu.sync_copy(oxv, ox_hbm.at[pl.ds(r0, nrows)])
        pltpu.sync_copy(oyv, oy_hbm.at[pl.ds(r0, nrows)])

    def _mkbufs(rows):
        return [pltpu.VMEM((rows,), jnp.float32) for _ in range(16)]

    @functools.partial(
        pl.kernel,
        out_type=(jax.ShapeDtypeStruct((n,), jnp.float32),
                  jax.ShapeDtypeStruct((n,), jnp.float32)),
        mesh=mesh,
        scratch_types=dict(
            bufs=_mkbufs(base),
            bufst=_mkbufs(max(tail, L)),
        ),
    )
    def combine_kernel(part_hbm, px_hbm, py_hbm, vx_hbm, vy_hbm,
                       ox_hbm, oy_hbm, bufs, bufst):
        cid = lax.axis_index("c")
        sid = lax.axis_index("s")
        wid = sid * NC + cid
        _combine_rows(part_hbm, px_hbm, py_hbm, vx_hbm, vy_hbm,
                      ox_hbm, oy_hbm, bufs, wid * base, base)

        @pl.when(wid == NW - 1)
        def _():
            if tail:
                _combine_rows(part_hbm, px_hbm, py_hbm, vx_hbm, vy_hbm,
                              ox_hbm, oy_hbm, bufst, NW * base, tail)

    return combine_kernel


def kernel(pos, vel, edge_index):
    n = pos.shape[0]
    e = edge_index.shape[1]
    px, py = pos[:, 0], pos[:, 1]
    vx, vy = vel[:, 0], vel[:, 1]
    src1d = edge_index[0]
    dst2d = edge_index[1].reshape(e // 128, 128)
    part = _edge_kernel(n, e)(px, py, vx, vy, src1d, dst2d)
    ox, oy = _combine_kernel(n)(part, px, py, vx, vy)
    return jnp.stack([ox, oy], axis=-1)
